# baseline (speedup 1.0000x reference)
import jax
import jax.numpy as jnp
from jax import lax
from jax.experimental import pallas as pl


def _matvec_block(x_ref, w_ref, b_ref, o_ref):
    x = x_ref[...]
    w = w_ref[...]
    d = x.shape[1]
    acc = lax.dot_general(
        w, x, (((1,), (1,)), ((), ())),
        preferred_element_type=jnp.float32)
    o_ref[...] = acc + b_ref[0, 0] * d


@jax.jit
def kernel(continuous, W_continuous, bias):
    n, d = continuous.shape
    out_dim = W_continuous.shape[1]
    w_row = W_continuous.T
    b2 = bias.reshape(1, 1)

    BN = 2048
    out = pl.pallas_call(
        _matvec_block,
        grid=(n // BN,),
        in_specs=[
            pl.BlockSpec((BN, d), lambda i: (i, 0)),
            pl.BlockSpec((1, d), lambda i: (0, 0)),
            pl.BlockSpec((1, 1), lambda i: (0, 0)),
        ],
        out_specs=pl.BlockSpec((1, BN), lambda i: (0, i)),
        out_shape=jax.ShapeDtypeStruct((1, n), jnp.float32),
    )(continuous, w_row, b2)
    return out.reshape(n, out_dim)


# speedup vs baseline: 1.1469x; 1.1469x over previous
import jax
import jax.numpy as jnp
from jax import lax
from jax.experimental import pallas as pl


def _matvec_block(x_ref, w_ref, b_ref, o_ref):
    x = x_ref[...]
    w = w_ref[...]
    d = x.shape[1]
    acc = lax.dot_general(
        w, x, (((1,), (1,)), ((), ())),
        preferred_element_type=jnp.float32)
    o_ref[...] = acc + b_ref[0, 0] * d


@jax.jit
def kernel(continuous, W_continuous, bias):
    n, d = continuous.shape
    out_dim = W_continuous.shape[1]
    w_row = W_continuous.T
    b2 = bias.reshape(1, 1)

    BN = 4096
    out = pl.pallas_call(
        _matvec_block,
        grid=(n // BN,),
        in_specs=[
            pl.BlockSpec((BN, d), lambda i: (i, 0)),
            pl.BlockSpec((1, d), lambda i: (0, 0)),
            pl.BlockSpec((1, 1), lambda i: (0, 0)),
        ],
        out_specs=pl.BlockSpec((1, BN), lambda i: (0, i)),
        out_shape=jax.ShapeDtypeStruct((1, n), jnp.float32),
    )(continuous, w_row, b2)
    return out.reshape(n, out_dim)


# speedup vs baseline: 1.1670x; 1.0175x over previous
import jax
import jax.numpy as jnp
from jax import lax
from jax.experimental import pallas as pl
from jax.experimental.pallas import tpu as pltpu

_NCH = 8


def _matvec_manual(x_hbm, w_ref, b_ref, o_ref, xv, sems):
    n, d = x_hbm.shape
    ch = n // _NCH
    for i in range(_NCH):
        pltpu.make_async_copy(
            x_hbm.at[pl.ds(i * ch, ch)], xv.at[pl.ds(i * ch, ch)], sems.at[i]
        ).start()
    w = w_ref[...]
    bias_term = b_ref[0, 0] * d
    for i in range(_NCH):
        pltpu.make_async_copy(
            x_hbm.at[pl.ds(i * ch, ch)], xv.at[pl.ds(i * ch, ch)], sems.at[i]
        ).wait()
        x = xv[pl.ds(i * ch, ch), :]
        acc = lax.dot_general(
            w, x, (((1,), (1,)), ((), ())),
            preferred_element_type=jnp.float32)
        o_ref[:, pl.ds(i * ch, ch)] = acc + bias_term


@jax.jit
def kernel(continuous, W_continuous, bias):
    n, d = continuous.shape
    out_dim = W_continuous.shape[1]
    w_row = W_continuous.T
    b2 = bias.reshape(1, 1)

    out = pl.pallas_call(
        _matvec_manual,
        grid=(1,),
        in_specs=[
            pl.BlockSpec(memory_space=pl.ANY),
            pl.BlockSpec((1, d), lambda i: (0, 0)),
            pl.BlockSpec((1, 1), lambda i: (0, 0)),
        ],
        out_specs=pl.BlockSpec((1, n), lambda i: (0, 0)),
        out_shape=jax.ShapeDtypeStruct((1, n), jnp.float32),
        scratch_shapes=[
            pltpu.VMEM((n, d), jnp.float32),
            pltpu.SemaphoreType.DMA((_NCH,)),
        ],
    )(continuous, w_row, b2)
    return out.reshape(n, out_dim)
